# baseline (speedup 1.0000x reference)
import jax
import jax.numpy as jnp
from jax import lax
from jax.experimental import pallas as pl
from jax.experimental.pallas import tpu as pltpu
from jax.experimental.pallas import tpu_sc as plsc

EMB_DIM = 64
NC = 2
NS = 16
NW = NC * NS


def _gather_body(table_hbm, idx_hbm, out_hbm, idx_v, sem):
    wid = lax.axis_index("s") * NC + lax.axis_index("c")
    b_per_w = idx_v.shape[0]
    base = wid * b_per_w
    pltpu.sync_copy(idx_hbm.at[pl.ds(base, b_per_w)], idx_v)
    lanes = lax.iota(jnp.int32, 16)

    def body(g, carry):
        v = idx_v[pl.ds(g * 16, 16)]
        for j in range(16):
            row = jnp.sum(jnp.where(lanes == j, v, 0))
            pltpu.async_copy(
                table_hbm.at[pl.ds(row, 1)],
                out_hbm.at[pl.ds(base + g * 16 + j, 1)],
                sem,
            )
        return carry

    lax.fori_loop(0, b_per_w // 16, body, 0)
    pltpu.make_async_copy(
        table_hbm.at[pl.ds(0, b_per_w)], out_hbm.at[pl.ds(base, b_per_w)], sem
    ).wait()


def _sc_gather(table, idx):
    n = idx.shape[0]
    b_per_w = n // NW
    mesh = plsc.VectorSubcoreMesh(core_axis_name="c", subcore_axis_name="s")
    k = pl.kernel(
        _gather_body,
        out_type=jax.ShapeDtypeStruct((n, EMB_DIM), jnp.float32),
        mesh=mesh,
        scratch_types=[
            pltpu.VMEM((b_per_w,), jnp.int32),
            pltpu.SemaphoreType.DMA,
        ],
        compiler_params=pltpu.CompilerParams(needs_layout_passes=False),
    )
    return k(table, idx)


def _mlp_body(eu_ref, ev_ref, w1u_ref, w1v_ref, b1_ref, w2_ref, b2_ref, out_ref):
    h = (
        jnp.dot(eu_ref[...], w1u_ref[...], preferred_element_type=jnp.float32)
        + jnp.dot(ev_ref[...], w1v_ref[...], preferred_element_type=jnp.float32)
        + b1_ref[...]
    )
    h = jnp.maximum(h, 0.0)
    o = jnp.dot(h, w2_ref[...], preferred_element_type=jnp.float32) + b2_ref[0, 0]
    out_ref[...] = jax.nn.sigmoid(o)


def _mlp(g, w1u, w1v, b1r, w2t, b2r, batch, blk):
    nb = batch // blk
    return pl.pallas_call(
        _mlp_body,
        grid=(nb,),
        in_specs=[
            pl.BlockSpec((blk, EMB_DIM), lambda i: (i, 0)),
            pl.BlockSpec((blk, EMB_DIM), lambda i, nb=nb: (i + nb, 0)),
            pl.BlockSpec((EMB_DIM, EMB_DIM), lambda i: (0, 0)),
            pl.BlockSpec((EMB_DIM, EMB_DIM), lambda i: (0, 0)),
            pl.BlockSpec((1, EMB_DIM), lambda i: (0, 0)),
            pl.BlockSpec((EMB_DIM, 1), lambda i: (0, 0)),
            pl.BlockSpec((1, 1), lambda i: (0, 0)),
        ],
        out_specs=pl.BlockSpec((blk, 1), lambda i: (i, 0)),
        out_shape=jax.ShapeDtypeStruct((batch, 1), jnp.float32),
    )(g, g, w1u, w1v, b1r, w2t, b2r)


def kernel(u_ids, v_ids, emb, W1, b1, W2, b2):
    batch = u_ids.shape[0]
    idx = jnp.concatenate([u_ids.astype(jnp.int32), v_ids.astype(jnp.int32)])
    g = _sc_gather(emb, idx)
    w1u = W1[:, :EMB_DIM].T
    w1v = W1[:, EMB_DIM:].T
    out = _mlp(
        g, w1u, w1v,
        b1.reshape(1, EMB_DIM), W2.T, b2.reshape(1, 1),
        batch, 1024,
    )
    return out[:, 0]


# speedup vs baseline: 2.2646x; 2.2646x over previous
import jax
import jax.numpy as jnp
from jax import lax
from jax.experimental import pallas as pl
from jax.experimental.pallas import tpu as pltpu
from jax.experimental.pallas import tpu_sc as plsc

EMB_DIM = 64
NC = 2
NS = 16
NW = NC * NS


HALF = 512


def _gather_body(table_hbm, idx_hbm, out_hbm, idx_v, rows_v, sem):
    wid = lax.axis_index("s") * NC + lax.axis_index("c")
    b_per_w = idx_v.shape[0]
    base = wid * b_per_w
    pltpu.sync_copy(idx_hbm.at[pl.ds(base, b_per_w)], idx_v)
    lanes = lax.iota(jnp.int32, 16)

    for h in range(b_per_w // HALF):
        def body(g, carry, h=h):
            o = pl.multiple_of(h * HALF + g * 16, 16)
            d = pl.multiple_of(g * 16, 16)
            v = idx_v[pl.ds(o, 16)]
            for j in range(16):
                row = jnp.sum(jnp.where(lanes == j, v, 0))
                pltpu.async_copy(
                    table_hbm.at[pl.ds(row, 1)], rows_v.at[pl.ds(d + j, 1)], sem
                )
            return carry

        lax.fori_loop(0, HALF // 16, body, 0)
        pltpu.make_async_copy(table_hbm.at[pl.ds(0, HALF)], rows_v, sem).wait()
        pltpu.sync_copy(rows_v, out_hbm.at[pl.ds(base + h * HALF, HALF)])


def _sc_gather(table, idx):
    n = idx.shape[0]
    b_per_w = n // NW
    mesh = plsc.VectorSubcoreMesh(core_axis_name="c", subcore_axis_name="s")
    k = pl.kernel(
        _gather_body,
        out_type=jax.ShapeDtypeStruct((n, EMB_DIM), jnp.float32),
        mesh=mesh,
        scratch_types=[
            pltpu.VMEM((b_per_w,), jnp.int32),
            pltpu.VMEM((HALF, EMB_DIM), jnp.float32),
            pltpu.SemaphoreType.DMA,
        ],
        compiler_params=pltpu.CompilerParams(needs_layout_passes=False),
    )
    return k(table, idx)


def _mlp_body(eu_ref, ev_ref, w1u_ref, w1v_ref, b1_ref, w2_ref, b2_ref, out_ref):
    h = (
        jnp.dot(eu_ref[...], w1u_ref[...], preferred_element_type=jnp.float32)
        + jnp.dot(ev_ref[...], w1v_ref[...], preferred_element_type=jnp.float32)
        + b1_ref[...]
    )
    h = jnp.maximum(h, 0.0)
    o = jnp.dot(h, w2_ref[...], preferred_element_type=jnp.float32) + b2_ref[0, 0]
    out_ref[...] = jax.nn.sigmoid(o)


def _mlp(g, w1u, w1v, b1r, w2t, b2r, batch, blk):
    nb = batch // blk
    return pl.pallas_call(
        _mlp_body,
        grid=(nb,),
        in_specs=[
            pl.BlockSpec((blk, EMB_DIM), lambda i: (i, 0)),
            pl.BlockSpec((blk, EMB_DIM), lambda i, nb=nb: (i + nb, 0)),
            pl.BlockSpec((EMB_DIM, EMB_DIM), lambda i: (0, 0)),
            pl.BlockSpec((EMB_DIM, EMB_DIM), lambda i: (0, 0)),
            pl.BlockSpec((1, EMB_DIM), lambda i: (0, 0)),
            pl.BlockSpec((EMB_DIM, 1), lambda i: (0, 0)),
            pl.BlockSpec((1, 1), lambda i: (0, 0)),
        ],
        out_specs=pl.BlockSpec((blk, 1), lambda i: (i, 0)),
        out_shape=jax.ShapeDtypeStruct((batch, 1), jnp.float32),
    )(g, g, w1u, w1v, b1r, w2t, b2r)


def kernel(u_ids, v_ids, emb, W1, b1, W2, b2):
    batch = u_ids.shape[0]
    idx = jnp.concatenate([u_ids.astype(jnp.int32), v_ids.astype(jnp.int32)])
    g = _sc_gather(emb, idx)
    w1u = W1[:, :EMB_DIM].T
    w1v = W1[:, EMB_DIM:].T
    out = _mlp(
        g, w1u, w1v,
        b1.reshape(1, EMB_DIM), W2.T, b2.reshape(1, 1),
        batch, 1024,
    )
    return out[:, 0]


# speedup vs baseline: 2.2771x; 1.0055x over previous
import jax
import jax.numpy as jnp
from jax import lax
from jax.experimental import pallas as pl
from jax.experimental.pallas import tpu as pltpu
from jax.experimental.pallas import tpu_sc as plsc

EMB_DIM = 64
NC = 2
NS = 16
NW = NC * NS


HALF = 512


def _gather_body(table_hbm, idx_hbm, out_hbm, idx_v, rows_v, sem):
    wid = lax.axis_index("s") * NC + lax.axis_index("c")
    b_per_w = idx_v.shape[0]
    base = wid * b_per_w
    pltpu.sync_copy(idx_hbm.at[pl.ds(base, b_per_w)], idx_v)

    for h in range(b_per_w // HALF):
        def body(g, carry, h=h):
            o = pl.multiple_of(h * HALF + g * 16, 16)
            d = pl.multiple_of(g * 16, 16)
            v = idx_v[pl.ds(o, 16)]
            for j in range(16):
                row = v[j]
                pltpu.async_copy(
                    table_hbm.at[pl.ds(row, 1)], rows_v.at[pl.ds(d + j, 1)], sem
                )
            return carry

        lax.fori_loop(0, HALF // 16, body, 0)
        pltpu.make_async_copy(table_hbm.at[pl.ds(0, HALF)], rows_v, sem).wait()
        pltpu.sync_copy(rows_v, out_hbm.at[pl.ds(base + h * HALF, HALF)])


def _sc_gather(table, idx):
    n = idx.shape[0]
    b_per_w = n // NW
    mesh = plsc.VectorSubcoreMesh(core_axis_name="c", subcore_axis_name="s")
    k = pl.kernel(
        _gather_body,
        out_type=jax.ShapeDtypeStruct((n, EMB_DIM), jnp.float32),
        mesh=mesh,
        scratch_types=[
            pltpu.VMEM((b_per_w,), jnp.int32),
            pltpu.VMEM((HALF, EMB_DIM), jnp.float32),
            pltpu.SemaphoreType.DMA,
        ],
    )
    return k(table, idx)


def _mlp_body(eu_ref, ev_ref, w1u_ref, w1v_ref, b1_ref, w2_ref, b2_ref, out_ref):
    h = (
        jnp.dot(eu_ref[...], w1u_ref[...], preferred_element_type=jnp.float32)
        + jnp.dot(ev_ref[...], w1v_ref[...], preferred_element_type=jnp.float32)
        + b1_ref[...]
    )
    h = jnp.maximum(h, 0.0)
    o = jnp.dot(h, w2_ref[...], preferred_element_type=jnp.float32) + b2_ref[0, 0]
    out_ref[...] = jax.nn.sigmoid(o)


def _mlp(g, w1u, w1v, b1r, w2t, b2r, batch, blk):
    nb = batch // blk
    return pl.pallas_call(
        _mlp_body,
        grid=(nb,),
        in_specs=[
            pl.BlockSpec((blk, EMB_DIM), lambda i: (i, 0)),
            pl.BlockSpec((blk, EMB_DIM), lambda i, nb=nb: (i + nb, 0)),
            pl.BlockSpec((EMB_DIM, EMB_DIM), lambda i: (0, 0)),
            pl.BlockSpec((EMB_DIM, EMB_DIM), lambda i: (0, 0)),
            pl.BlockSpec((1, EMB_DIM), lambda i: (0, 0)),
            pl.BlockSpec((EMB_DIM, 1), lambda i: (0, 0)),
            pl.BlockSpec((1, 1), lambda i: (0, 0)),
        ],
        out_specs=pl.BlockSpec((blk, 1), lambda i: (i, 0)),
        out_shape=jax.ShapeDtypeStruct((batch, 1), jnp.float32),
    )(g, g, w1u, w1v, b1r, w2t, b2r)


def kernel(u_ids, v_ids, emb, W1, b1, W2, b2):
    batch = u_ids.shape[0]
    idx = jnp.concatenate([u_ids.astype(jnp.int32), v_ids.astype(jnp.int32)])
    g = _sc_gather(emb, idx)
    w1u = W1[:, :EMB_DIM].T
    w1v = W1[:, EMB_DIM:].T
    out = _mlp(
        g, w1u, w1v,
        b1.reshape(1, EMB_DIM), W2.T, b2.reshape(1, 1),
        batch, 1024,
    )
    return out[:, 0]
